# baseline (speedup 1.0000x reference)
import functools

import jax
import jax.numpy as jnp
from jax import lax
from jax.experimental import pallas as pl
from jax.experimental.pallas import tpu as pltpu
from jax.experimental.pallas import tpu_sc as plsc

NC = 2
NS = 16
NW = NC * NS
C = 128
D = 64


@functools.partial(jax.jit, static_argnums=(2,))
def _gather(embed, idx, n_total):
    n_per_w = n_total // NW
    nch = n_per_w // C
    mesh = plsc.VectorSubcoreMesh(core_axis_name="c", subcore_axis_name="s")

    @functools.partial(
        pl.kernel,
        mesh=mesh,
        out_type=jax.ShapeDtypeStruct((n_total, D), jnp.float32),
        scratch_types=[
            pltpu.VMEM((nch, C), jnp.int32),
            pltpu.VMEM((C, D), jnp.float32),
            pltpu.SemaphoreType.DMA,
        ],
        compiler_params=pltpu.CompilerParams(use_tc_tiling_on_sc=False),
    )
    def gather_kernel(table_hbm, idx_hbm, out_hbm, idx_v, rows_v, gsem):
        wid = lax.axis_index("s") * NC + lax.axis_index("c")
        base = wid * n_per_w
        pltpu.sync_copy(idx_hbm.at[wid], idx_v)

        @pl.loop(0, nch)
        def _chunk(j):
            pltpu.async_copy(table_hbm.at[idx_v.at[j]], rows_v, gsem).wait()
            pltpu.sync_copy(rows_v, out_hbm.at[pl.ds(base + j * C, C)])

    return gather_kernel(embed, idx)


def kernel(x, embed):
    b, h = x.shape
    n_total = b * h
    idx = x.astype(jnp.int32).reshape(NW, n_total // (NW * C), C)
    out = _gather(embed, idx, n_total)
    return out.reshape(b, h, D)


# speedup vs baseline: 1.0618x; 1.0618x over previous
import functools

import jax
import jax.numpy as jnp
from jax import lax
from jax.experimental import pallas as pl
from jax.experimental.pallas import tpu as pltpu
from jax.experimental.pallas import tpu_sc as plsc

NC = 2
NS = 16
NW = NC * NS
C = 128
D = 64


K = 4


@functools.partial(jax.jit, static_argnums=(2,))
def _gather(embed, idx, n_total):
    n_per_w = n_total // NW
    nch = n_per_w // C
    nhalf = nch // K
    assert nch % K == 0 and nhalf % 2 == 0
    mesh = plsc.VectorSubcoreMesh(core_axis_name="c", subcore_axis_name="s")

    @functools.partial(
        pl.kernel,
        mesh=mesh,
        out_type=jax.ShapeDtypeStruct((n_total, D), jnp.float32),
        scratch_types=[
            pltpu.VMEM((nch, C), jnp.int32),
            pltpu.VMEM((2, K, C, D), jnp.float32),
            pltpu.SemaphoreType.DMA,
            pltpu.SemaphoreType.DMA,
            pltpu.SemaphoreType.DMA,
        ],
        compiler_params=pltpu.CompilerParams(use_tc_tiling_on_sc=False),
    )
    def gather_kernel(table_hbm, idx_hbm, out_hbm, idx_v, rows_v, gsem, ssem0, ssem1):
        wid = lax.axis_index("s") * NC + lax.axis_index("c")
        base = wid * n_per_w
        ssems = (ssem0, ssem1)
        pltpu.sync_copy(idx_hbm.at[wid], idx_v)

        for b in range(K):
            pltpu.async_copy(table_hbm.at[idx_v.at[b]], rows_v.at[0, b], gsem)

        @pl.loop(0, nhalf, step=2)
        def _half(h0):
            for p in range(2):
                h = h0 + p
                q = 1 - p
                jj = h * K
                for b in range(K):
                    pltpu.make_async_copy(
                        table_hbm.at[idx_v.at[jj + b]], rows_v.at[p, b], gsem
                    ).wait()
                for b in range(K):
                    pltpu.async_copy(
                        rows_v.at[p, b],
                        out_hbm.at[pl.ds(base + (jj + b) * C, C)],
                        ssems[p],
                    )

                @pl.when(h > 0)
                def _():
                    for b in range(K):
                        pltpu.make_async_copy(
                            rows_v.at[q, b], out_hbm.at[pl.ds(base, C)], ssems[q]
                        ).wait()

                @pl.when(h + 1 < nhalf)
                def _():
                    njj = (h + 1) * K
                    for b in range(K):
                        pltpu.async_copy(
                            table_hbm.at[idx_v.at[njj + b]], rows_v.at[q, b], gsem
                        )

        last = (nhalf - 1) % 2
        for b in range(K):
            pltpu.make_async_copy(
                rows_v.at[last, b], out_hbm.at[pl.ds(base, C)], ssems[last]
            ).wait()

    return gather_kernel(embed, idx)


def kernel(x, embed):
    b, h = x.shape
    n_total = b * h
    idx = x.astype(jnp.int32).reshape(NW, n_total // (NW * C), C)
    out = _gather(embed, idx, n_total)
    return out.reshape(b, h, D)
